# baseline (speedup 1.0000x reference)
import functools

import jax
import jax.numpy as jnp
import numpy as np
from jax import lax
from jax.experimental import pallas as pl
from jax.experimental.pallas import tpu as pltpu
from jax.experimental.pallas import tpu_sc as plsc

N_NODES = 10000
D = 128
D2 = 256
N_GRAPHS = 64
EPS_MSG = 1e-7
BN_EPS = 1e-5
BLK = 1000

NC, NS, L = 2, 16, 16
CHUNK = 128
CPT = 160
E_PAD = NS * CPT * CHUNK
NROWS = 10240
RPT = NROWS // NS


def _tc_edge_feat_body(ea_ref, w_ref, b_ref, out_ref):
    out_ref[...] = jnp.dot(ea_ref[...], w_ref[...],
                           preferred_element_type=jnp.float32) + b_ref[...]


def _tc_edge_feat(ea2d, tWe, tbe):
    eblk = 8192
    grid = (E_PAD // eblk,)
    return pl.pallas_call(
        _tc_edge_feat_body,
        grid=grid,
        in_specs=[pl.BlockSpec((eblk, 4), lambda i: (i, 0)),
                  pl.BlockSpec((4, D), lambda i: (0, 0)),
                  pl.BlockSpec((1, D), lambda i: (0, 0))],
        out_specs=pl.BlockSpec((eblk, D), lambda i: (i, 0)),
        out_shape=jax.ShapeDtypeStruct((E_PAD, D), jnp.float32),
    )(ea2d, tWe, tbe)


def _tc_layer_body(s1_ref, s2_ref, tinv_ref, g_ref, xres_ref, w1_ref, b1_ref,
                   w2_ref, b2_ref, nw_ref, nb_ref, x_out_ref, g_out_ref, *,
                   has_res):
    out = s2_ref[...] * tinv_ref[...] / (s1_ref[...] + 1e-16) + g_ref[...]
    h = jnp.dot(out, w1_ref[...], preferred_element_type=jnp.float32) + b1_ref[...]
    h = jnp.maximum(h, 0.0)
    h = jnp.dot(h, w2_ref[...], preferred_element_type=jnp.float32) + b2_ref[...]
    if has_res:
        h = h + xres_ref[...]
    x_out_ref[...] = h
    gnext = h * nw_ref[...] + nb_ref[...]
    g_out_ref[...] = jnp.where(gnext >= 0.0, gnext, 0.01 * gnext)


def _tc_layer(s1, s2, tinv, g, xres, w1, b1, w2, b2, nw, nb, has_res):
    grid = (N_NODES // BLK,)
    row = pl.BlockSpec((BLK, D), lambda i: (i, 0))
    full = lambda shape: pl.BlockSpec(shape, lambda i: (0,) * len(shape))
    return pl.pallas_call(
        functools.partial(_tc_layer_body, has_res=has_res),
        grid=grid,
        in_specs=[row, row, full((1, D)), row, row, full((D, D2)),
                  full((1, D2)), full((D2, D)), full((1, D)), full((1, D)),
                  full((1, D))],
        out_specs=[row, row],
        out_shape=[jax.ShapeDtypeStruct((N_NODES, D), jnp.float32)] * 2,
    )(s1, s2, tinv, g, xres, w1, b1, w2, b2, nw, nb)


def _tc_final_body(s1_ref, s2_ref, tinv_ref, g_ref, xres_ref, w1_ref, b1_ref,
                   w2_ref, b2_ref, nw_ref, nb_ref, batch_ref, out_ref, acc_ref,
                   cnt_ref):
    i = pl.program_id(0)

    @pl.when(i == 0)
    def _():
        acc_ref[...] = jnp.zeros_like(acc_ref)
        cnt_ref[...] = jnp.zeros_like(cnt_ref)

    out = s2_ref[...] * tinv_ref[...] / (s1_ref[...] + 1e-16) + g_ref[...]
    h = jnp.dot(out, w1_ref[...], preferred_element_type=jnp.float32) + b1_ref[...]
    h = jnp.maximum(h, 0.0)
    h = jnp.dot(h, w2_ref[...], preferred_element_type=jnp.float32) + b2_ref[...]
    h = h + xres_ref[...]
    y = h * nw_ref[...] + nb_ref[...]
    y = jnp.where(y >= 0.0, y, 0.01 * y)
    gids = lax.broadcasted_iota(jnp.int32, (N_GRAPHS, BLK), 0)
    onehot = (batch_ref[0] == gids).astype(jnp.float32)
    acc_ref[...] += jnp.dot(onehot, y, preferred_element_type=jnp.float32)
    cnt_ref[...] += jnp.sum(onehot, axis=1, keepdims=True)

    @pl.when(i == pl.num_programs(0) - 1)
    def _():
        out_ref[...] = acc_ref[...] / jnp.maximum(cnt_ref[...], 1.0)


def _tc_final(s1, s2, tinv, g, xres, w1, b1, w2, b2, nw, nb, batch3d):
    grid = (N_NODES // BLK,)
    row = pl.BlockSpec((BLK, D), lambda i: (i, 0))
    full = lambda shape: pl.BlockSpec(shape, lambda i: (0,) * len(shape))
    return pl.pallas_call(
        _tc_final_body,
        grid=grid,
        in_specs=[row, row, full((1, D)), row, row, full((D, D2)),
                  full((1, D2)), full((D2, D)), full((1, D)), full((1, D)),
                  full((1, D)), pl.BlockSpec((1, 1, BLK), lambda i: (i, 0, 0))],
        out_specs=full((N_GRAPHS, D)),
        out_shape=jax.ShapeDtypeStruct((N_GRAPHS, D), jnp.float32),
        scratch_shapes=[pltpu.VMEM((N_GRAPHS, D), jnp.float32),
                        pltpu.VMEM((N_GRAPHS, 1), jnp.float32)],
    )(s1, s2, tinv, g, xres, w1, b1, w2, b2, nw, nb, batch3d)


def _sc_edge_body(g_hbm, src_hbm, dst_hbm, ea_hbm, pp_hbm, out_hbm,
                  src_v, dst_v, ea_v, rows_v, buf_v, pp_v, sem, acc_sh):
    c = lax.axis_index("c")
    t = lax.axis_index("s")
    pltpu.sync_copy(pp_hbm, pp_v)

    zero = jnp.zeros((L,), jnp.float32)

    def zb(j, _):
        for s in range(8):
            buf_v[j, pl.ds(16 * s, 16)] = zero
        return 0
    lax.fori_loop(0, CHUNK, zb, 0)

    def zs(k, _):
        pltpu.sync_copy(buf_v, acc_sh.at[pl.ds(t * RPT + k * CHUNK, CHUNK)])
        return 0
    lax.fori_loop(0, RPT // CHUNK, zs, 0)
    plsc.subcore_barrier()

    we = [[pp_v[k, pl.ds(16 * s, 16)] for s in range(8)] for k in range(4)]
    bev = [pp_v[4, pl.ds(16 * s, 16)] for s in range(8)]
    pv = pp_v[5, pl.ds(0, 16)]
    t_sc = pv[0]
    teps = pv[1]
    cf = jnp.full((L,), c, jnp.int32).astype(jnp.float32)
    s0v = 1.0 - cf
    s1v = cf

    def chunk_body(k, _):
        ck = t * CPT + k
        pltpu.sync_copy(src_hbm.at[ck], src_v)
        pltpu.sync_copy(dst_hbm.at[ck], dst_v)
        pltpu.sync_copy(ea_hbm.at[ck], ea_v.at[pl.ds(0, 4 * CHUNK)])
        pltpu.async_copy(g_hbm.at[src_v], rows_v, sem).wait()

        def edge_body(j, _):
            av = ea_v[pl.ds(4 * j, 16)]
            a0 = av[0]
            a1 = av[1]
            a2 = av[2]
            a3 = av[3]
            for s in range(8):
                sl = pl.ds(16 * s, 16)
                ev = a0 * we[0][s] + a1 * we[1][s] + a2 * we[2][s] \
                    + a3 * we[3][s] + bev[s]
                u = jnp.maximum(t_sc * rows_v[j, sl] + ev, 0.0) + teps
                w = jnp.exp(u)
                buf_v[j, sl] = w * (s0v + s1v * u)
            return 0
        lax.fori_loop(0, CHUNK, edge_body, 0)
        pltpu.sync_copy(buf_v, acc_sh.at[dst_v], add=True)
        return 0
    lax.fori_loop(0, CPT, chunk_body, 0)
    plsc.subcore_barrier()

    def wb(k, _):
        r = t * RPT + k * CHUNK
        pltpu.sync_copy(acc_sh.at[pl.ds(r, CHUNK)], buf_v)
        pltpu.sync_copy(buf_v, out_hbm.at[pl.ds(c * NROWS + r, CHUNK)])
        return 0
    lax.fori_loop(0, RPT // CHUNK, wb, 0)


def _sc_edge(g, src2, dst2, ea2, pp):
    f32 = jnp.float32
    mesh = plsc.VectorSubcoreMesh(core_axis_name="c", subcore_axis_name="s",
                                  num_cores=NC, num_subcores=NS)
    kern = pl.kernel(
        _sc_edge_body,
        out_type=jax.ShapeDtypeStruct((2 * NROWS, D), f32),
        mesh=mesh,
        scratch_types=[
            pltpu.VMEM((CHUNK,), jnp.int32),
            pltpu.VMEM((CHUNK,), jnp.int32),
            pltpu.VMEM((4 * CHUNK + 16,), f32),
            pltpu.VMEM((CHUNK, D), f32),
            pltpu.VMEM((CHUNK, D), f32),
            pltpu.VMEM((6, D), f32),
            pltpu.SemaphoreType.DMA,
            pltpu.VMEM_SHARED((NROWS, D), f32),
        ],
    )
    res = kern(g, src2, dst2, ea2, pp)
    return res[:N_NODES], res[NROWS:NROWS + N_NODES]


def _edge_phase(g, src2, dst2, ea2, We, be, t):
    scal = jnp.concatenate([jnp.full((1, 1), t, jnp.float32),
                            jnp.full((1, 1), t * EPS_MSG, jnp.float32),
                            jnp.zeros((1, D - 2), jnp.float32)], axis=1)
    pp = jnp.concatenate([t * We, (t * be)[None, :], scal], axis=0)
    return _sc_edge(g, src2, dst2, ea2, pp)


def kernel(x, edge_index, edge_attr, batch, clinical, params):
    del clinical
    src, dst = edge_index[0], edge_index[1]
    n_edges = src.shape[0]
    npad = E_PAD - n_edges
    src2 = jnp.concatenate([src.astype(jnp.int32),
                            jnp.zeros((npad,), jnp.int32)]).reshape(NS * CPT, CHUNK)
    dst2 = jnp.concatenate([dst.astype(jnp.int32),
                            jnp.full((npad,), N_NODES, jnp.int32)]).reshape(NS * CPT, CHUNK)
    ea2 = jnp.concatenate([edge_attr.astype(jnp.float32),
                           jnp.zeros((npad, 4), jnp.float32)]).reshape(
                               NS * CPT, 4 * CHUNK)
    bns = 1.0 / np.sqrt(1.0 + BN_EPS)

    def folded(i):
        p = params[f"conv{i}"]
        s = p["bn1_w"] * bns
        w1 = p["W1"] * s[None, :]
        b1 = (p["b1"] * s + p["bn1_b"])[None, :]
        w2 = p["W2"]
        b2 = p["b2"][None, :]
        return w1, b1, w2, b2

    def norm(name):
        nm = params[name]
        return (nm["w"] * bns)[None, :], nm["b"][None, :]

    nw1, nb1 = norm("norm1")
    nw2, nb2 = norm("norm2")
    nw0, nb0 = norm("norm0")
    batch3d = batch.astype(jnp.int32).reshape(N_NODES // BLK, 1, BLK)

    tinvs = [jnp.full((1, D), 1.0, jnp.float32) / params[f"conv{i}"]["t"]
             for i in range(3)]

    p0 = params["conv0"]
    s1, s2 = _edge_phase(x, src2, dst2, ea2, p0["We"], p0["be"], p0["t"])
    x1, g1 = _tc_layer(s1, s2, tinvs[0], x, x, *folded(0), nw1, nb1,
                       has_res=False)
    p1 = params["conv1"]
    s1, s2 = _edge_phase(g1, src2, dst2, ea2, p1["We"], p1["be"], p1["t"])
    x2, g2 = _tc_layer(s1, s2, tinvs[1], g1, x1, *folded(1), nw2, nb2,
                       has_res=True)
    p2 = params["conv2"]
    s1, s2 = _edge_phase(g2, src2, dst2, ea2, p2["We"], p2["be"], p2["t"])
    return _tc_final(s1, s2, tinvs[2], g2, x2, *folded(2), nw0, nb0, batch3d)


# speedup vs baseline: 1.1652x; 1.1652x over previous
import functools

import jax
import jax.numpy as jnp
import numpy as np
from jax import lax
from jax.experimental import pallas as pl
from jax.experimental.pallas import tpu as pltpu
from jax.experimental.pallas import tpu_sc as plsc

N_NODES = 10000
D = 128
D2 = 256
N_GRAPHS = 64
EPS_MSG = 1e-7
BN_EPS = 1e-5
BLK = 1000

NC, NS, L = 2, 16, 16
CHUNK = 112
CPT = 184
E_PAD = NS * CPT * CHUNK
NROWS = 10240
RPT = NROWS // NS
ZCH = 80
EA_W = 512


def _tc_edge_feat_body(ea_ref, w_ref, b_ref, out_ref):
    out_ref[...] = jnp.dot(ea_ref[...], w_ref[...],
                           preferred_element_type=jnp.float32) + b_ref[...]


def _tc_edge_feat(ea2d, tWe, tbe):
    eblk = 8192
    grid = (E_PAD // eblk,)
    return pl.pallas_call(
        _tc_edge_feat_body,
        grid=grid,
        in_specs=[pl.BlockSpec((eblk, 4), lambda i: (i, 0)),
                  pl.BlockSpec((4, D), lambda i: (0, 0)),
                  pl.BlockSpec((1, D), lambda i: (0, 0))],
        out_specs=pl.BlockSpec((eblk, D), lambda i: (i, 0)),
        out_shape=jax.ShapeDtypeStruct((E_PAD, D), jnp.float32),
    )(ea2d, tWe, tbe)


def _tc_layer_body(s1_ref, s2_ref, tinv_ref, g_ref, xres_ref, w1_ref, b1_ref,
                   w2_ref, b2_ref, nw_ref, nb_ref, x_out_ref, g_out_ref, *,
                   has_res):
    out = s2_ref[...] * tinv_ref[...] / (s1_ref[...] + 1e-16) + g_ref[...]
    h = jnp.dot(out, w1_ref[...], preferred_element_type=jnp.float32) + b1_ref[...]
    h = jnp.maximum(h, 0.0)
    h = jnp.dot(h, w2_ref[...], preferred_element_type=jnp.float32) + b2_ref[...]
    if has_res:
        h = h + xres_ref[...]
    x_out_ref[...] = h
    gnext = h * nw_ref[...] + nb_ref[...]
    g_out_ref[...] = jnp.where(gnext >= 0.0, gnext, 0.01 * gnext)


def _tc_layer(s1, s2, tinv, g, xres, w1, b1, w2, b2, nw, nb, has_res):
    grid = (N_NODES // BLK,)
    row = pl.BlockSpec((BLK, D), lambda i: (i, 0))
    full = lambda shape: pl.BlockSpec(shape, lambda i: (0,) * len(shape))
    return pl.pallas_call(
        functools.partial(_tc_layer_body, has_res=has_res),
        grid=grid,
        in_specs=[row, row, full((1, D)), row, row, full((D, D2)),
                  full((1, D2)), full((D2, D)), full((1, D)), full((1, D)),
                  full((1, D))],
        out_specs=[row, row],
        out_shape=[jax.ShapeDtypeStruct((N_NODES, D), jnp.float32)] * 2,
    )(s1, s2, tinv, g, xres, w1, b1, w2, b2, nw, nb)


def _tc_final_body(s1_ref, s2_ref, tinv_ref, g_ref, xres_ref, w1_ref, b1_ref,
                   w2_ref, b2_ref, nw_ref, nb_ref, batch_ref, out_ref, acc_ref,
                   cnt_ref):
    i = pl.program_id(0)

    @pl.when(i == 0)
    def _():
        acc_ref[...] = jnp.zeros_like(acc_ref)
        cnt_ref[...] = jnp.zeros_like(cnt_ref)

    out = s2_ref[...] * tinv_ref[...] / (s1_ref[...] + 1e-16) + g_ref[...]
    h = jnp.dot(out, w1_ref[...], preferred_element_type=jnp.float32) + b1_ref[...]
    h = jnp.maximum(h, 0.0)
    h = jnp.dot(h, w2_ref[...], preferred_element_type=jnp.float32) + b2_ref[...]
    h = h + xres_ref[...]
    y = h * nw_ref[...] + nb_ref[...]
    y = jnp.where(y >= 0.0, y, 0.01 * y)
    gids = lax.broadcasted_iota(jnp.int32, (N_GRAPHS, BLK), 0)
    onehot = (batch_ref[0] == gids).astype(jnp.float32)
    acc_ref[...] += jnp.dot(onehot, y, preferred_element_type=jnp.float32)
    cnt_ref[...] += jnp.sum(onehot, axis=1, keepdims=True)

    @pl.when(i == pl.num_programs(0) - 1)
    def _():
        out_ref[...] = acc_ref[...] / jnp.maximum(cnt_ref[...], 1.0)


def _tc_final(s1, s2, tinv, g, xres, w1, b1, w2, b2, nw, nb, batch3d):
    grid = (N_NODES // BLK,)
    row = pl.BlockSpec((BLK, D), lambda i: (i, 0))
    full = lambda shape: pl.BlockSpec(shape, lambda i: (0,) * len(shape))
    return pl.pallas_call(
        _tc_final_body,
        grid=grid,
        in_specs=[row, row, full((1, D)), row, row, full((D, D2)),
                  full((1, D2)), full((D2, D)), full((1, D)), full((1, D)),
                  full((1, D)), pl.BlockSpec((1, 1, BLK), lambda i: (i, 0, 0))],
        out_specs=full((N_GRAPHS, D)),
        out_shape=jax.ShapeDtypeStruct((N_GRAPHS, D), jnp.float32),
        scratch_shapes=[pltpu.VMEM((N_GRAPHS, D), jnp.float32),
                        pltpu.VMEM((N_GRAPHS, 1), jnp.float32)],
    )(s1, s2, tinv, g, xres, w1, b1, w2, b2, nw, nb, batch3d)


def _sc_edge_body(g_hbm, src_hbm, dst_hbm, ea_hbm, pp_hbm, out_hbm,
                  src_v, src_b, dst_v, ea_v, rows_v, rows_b, buf_v, pp_v,
                  sem, sem_b, acc_sh):
    c = lax.axis_index("c")
    t = lax.axis_index("s")
    pltpu.sync_copy(pp_hbm, pp_v)

    zero = jnp.zeros((L,), jnp.float32)

    def zb(j, _):
        for s in range(8):
            buf_v[j, pl.ds(16 * s, 16)] = zero
        return 0
    lax.fori_loop(0, CHUNK, zb, 0)

    def zs(k, _):
        pltpu.sync_copy(buf_v.at[pl.ds(0, ZCH)],
                        acc_sh.at[pl.ds(t * RPT + k * ZCH, ZCH)])
        return 0
    lax.fori_loop(0, RPT // ZCH, zs, 0)
    plsc.subcore_barrier()

    we = [[pp_v[k, pl.ds(16 * s, 16)] for s in range(8)] for k in range(4)]
    bev = [pp_v[4, pl.ds(16 * s, 16)] for s in range(8)]
    pv = pp_v[5, pl.ds(0, 16)]
    t_sc = pv[0]
    teps = pv[1]
    cf = jnp.full((L,), c, jnp.int32).astype(jnp.float32)
    s0v = 1.0 - cf
    s1v = cf

    def start_gather(k, src_ref, rows_ref, semx):
        pltpu.sync_copy(src_hbm.at[t * CPT + k], src_ref)
        pltpu.async_copy(g_hbm.at[src_ref], rows_ref, semx)

    def wait_gather(src_ref, rows_ref, semx):
        pltpu.make_async_copy(g_hbm.at[src_ref], rows_ref, semx).wait()

    def compute_chunk(k, rows_ref):
        ck = t * CPT + k
        pltpu.sync_copy(dst_hbm.at[ck], dst_v)
        pltpu.sync_copy(ea_hbm.at[ck], ea_v.at[pl.ds(0, EA_W)])

        def edge_body(j, _):
            av = ea_v[pl.ds(4 * j, 16)]
            a0 = av[0]
            a1 = av[1]
            a2 = av[2]
            a3 = av[3]
            for s in range(8):
                sl = pl.ds(16 * s, 16)
                ev = a0 * we[0][s] + a1 * we[1][s] + a2 * we[2][s] \
                    + a3 * we[3][s] + bev[s]
                u = jnp.maximum(t_sc * rows_ref[j, sl] + ev, 0.0) + teps
                w = jnp.exp(u)
                buf_v[j, sl] = w * (s0v + s1v * u)
            return 0
        lax.fori_loop(0, CHUNK, edge_body, 0)
        pltpu.sync_copy(buf_v, acc_sh.at[dst_v], add=True)

    start_gather(0, src_v, rows_v, sem)

    def pair_body(i, _):
        k0 = 2 * i
        wait_gather(src_v, rows_v, sem)
        start_gather(k0 + 1, src_b, rows_b, sem_b)
        compute_chunk(k0, rows_v)
        wait_gather(src_b, rows_b, sem_b)

        @pl.when(i < CPT // 2 - 1)
        def _():
            start_gather(k0 + 2, src_v, rows_v, sem)

        compute_chunk(k0 + 1, rows_b)
        return 0
    lax.fori_loop(0, CPT // 2, pair_body, 0)
    plsc.subcore_barrier()

    def wb(k, _):
        r = t * RPT + k * ZCH
        pltpu.sync_copy(acc_sh.at[pl.ds(r, ZCH)], buf_v.at[pl.ds(0, ZCH)])
        pltpu.sync_copy(buf_v.at[pl.ds(0, ZCH)],
                        out_hbm.at[pl.ds(c * NROWS + r, ZCH)])
        return 0
    lax.fori_loop(0, RPT // ZCH, wb, 0)


def _sc_edge(g, src2, dst2, ea2, pp):
    f32 = jnp.float32
    mesh = plsc.VectorSubcoreMesh(core_axis_name="c", subcore_axis_name="s",
                                  num_cores=NC, num_subcores=NS)
    kern = pl.kernel(
        _sc_edge_body,
        out_type=jax.ShapeDtypeStruct((2 * NROWS, D), f32),
        mesh=mesh,
        scratch_types=[
            pltpu.VMEM((CHUNK,), jnp.int32),
            pltpu.VMEM((CHUNK,), jnp.int32),
            pltpu.VMEM((CHUNK,), jnp.int32),
            pltpu.VMEM((EA_W + 16,), f32),
            pltpu.VMEM((CHUNK, D), f32),
            pltpu.VMEM((CHUNK, D), f32),
            pltpu.VMEM((CHUNK, D), f32),
            pltpu.VMEM((6, D), f32),
            pltpu.SemaphoreType.DMA,
            pltpu.SemaphoreType.DMA,
            pltpu.VMEM_SHARED((NROWS, D), f32),
        ],
    )
    res = kern(g, src2, dst2, ea2, pp)
    return res[:N_NODES], res[NROWS:NROWS + N_NODES]


def _edge_phase(g, src2, dst2, ea2, We, be, t):
    scal = jnp.concatenate([jnp.full((1, 1), t, jnp.float32),
                            jnp.full((1, 1), t * EPS_MSG, jnp.float32),
                            jnp.zeros((1, D - 2), jnp.float32)], axis=1)
    pp = jnp.concatenate([t * We, (t * be)[None, :], scal], axis=0)
    return _sc_edge(g, src2, dst2, ea2, pp)


def kernel(x, edge_index, edge_attr, batch, clinical, params):
    del clinical
    src, dst = edge_index[0], edge_index[1]
    n_edges = src.shape[0]
    npad = E_PAD - n_edges
    src2 = jnp.concatenate([src.astype(jnp.int32),
                            jnp.zeros((npad,), jnp.int32)]).reshape(NS * CPT, CHUNK)
    dst2 = jnp.concatenate([dst.astype(jnp.int32),
                            jnp.full((npad,), N_NODES, jnp.int32)]).reshape(NS * CPT, CHUNK)
    ea2 = jnp.concatenate([edge_attr.astype(jnp.float32),
                           jnp.zeros((npad, 4), jnp.float32)]).reshape(
                               NS * CPT, 4 * CHUNK)
    ea2 = jnp.concatenate(
        [ea2, jnp.zeros((NS * CPT, EA_W - 4 * CHUNK), jnp.float32)], axis=1)
    bns = 1.0 / np.sqrt(1.0 + BN_EPS)

    def folded(i):
        p = params[f"conv{i}"]
        s = p["bn1_w"] * bns
        w1 = p["W1"] * s[None, :]
        b1 = (p["b1"] * s + p["bn1_b"])[None, :]
        w2 = p["W2"]
        b2 = p["b2"][None, :]
        return w1, b1, w2, b2

    def norm(name):
        nm = params[name]
        return (nm["w"] * bns)[None, :], nm["b"][None, :]

    nw1, nb1 = norm("norm1")
    nw2, nb2 = norm("norm2")
    nw0, nb0 = norm("norm0")
    batch3d = batch.astype(jnp.int32).reshape(N_NODES // BLK, 1, BLK)

    tinvs = [jnp.full((1, D), 1.0, jnp.float32) / params[f"conv{i}"]["t"]
             for i in range(3)]

    p0 = params["conv0"]
    s1, s2 = _edge_phase(x, src2, dst2, ea2, p0["We"], p0["be"], p0["t"])
    x1, g1 = _tc_layer(s1, s2, tinvs[0], x, x, *folded(0), nw1, nb1,
                       has_res=False)
    p1 = params["conv1"]
    s1, s2 = _edge_phase(g1, src2, dst2, ea2, p1["We"], p1["be"], p1["t"])
    x2, g2 = _tc_layer(s1, s2, tinvs[1], g1, x1, *folded(1), nw2, nb2,
                       has_res=True)
    p2 = params["conv2"]
    s1, s2 = _edge_phase(g2, src2, dst2, ea2, p2["We"], p2["be"], p2["t"])
    return _tc_final(s1, s2, tinvs[2], g2, x2, *folded(2), nw0, nb0, batch3d)
